# baseline (speedup 1.0000x reference)
import functools

import jax
import jax.numpy as jnp
from jax import lax
from jax.experimental import pallas as pl
from jax.experimental.pallas import tpu as pltpu, tpu_sc as plsc

R = 8
N = 10000
F = 300
FP = 384
RW = R * FP
E = 160000
NG = 64
NL = 4
EPS = 1e-5
M = R * N

CH = 16
CPT = 20
NTILES = 32
NCH = NTILES * CPT
NP = NCH * CH
GSZ = 16

NB = 25
BN_ = N // NB


REG = 24
EE = E + N
PE = EE + NCH * GSZ
SL = RW // 128
GG = 8


def _sc_segsum(h2, src_p, sct_p, eo2, zbuf):
    mesh = plsc.VectorSubcoreMesh(core_axis_name="c", subcore_axis_name="s")

    @functools.partial(
        pl.kernel,
        out_type=jax.ShapeDtypeStruct((NP, SL, 128), jnp.float32),
        mesh=mesh,
        scratch_types=[
            pltpu.VMEM((16,), jnp.int32),
            pltpu.VMEM((GG,), jnp.int32),
            pltpu.VMEM((GG,), jnp.int32),
            pltpu.VMEM((GG,), jnp.int32),
            pltpu.VMEM((GG,), jnp.int32),
            pltpu.VMEM((GG, SL, 128), jnp.float32),
            pltpu.VMEM((GG, SL, 128), jnp.float32),
            pltpu.VMEM_SHARED((16 * REG, SL, 128), jnp.float32),
            pltpu.SemaphoreType.DMA,
            pltpu.SemaphoreType.DMA,
            pltpu.SemaphoreType.DMA,
        ],
    )
    def k(h_hbm, src_hbm, sct_hbm, eo_hbm, z_hbm, out_hbm,
          eo_v, idx0, idx1, sct0, sct1, land0, land1, acc_s,
          si0, sg0, ss0):
        si1, sg1, ss1 = si0, sg0, ss0
        sid = lax.axis_index("s")
        wid = sid * 2 + lax.axis_index("c")
        rbase = sid * REG
        bufs = ((idx0, sct0, land0, si0, sg0, ss0),
                (idx1, sct1, land1, si1, sg1, ss1))

        def cbody(c, _):
            chunk = wid * CPT + c
            pltpu.sync_copy(eo_hbm.at[chunk], eo_v)
            eo_vec = eo_v[...]
            lo = pl.multiple_of(eo_vec[0], GSZ)
            hi = eo_vec[1]
            pltpu.sync_copy(z_hbm, acc_s.at[pl.ds(rbase, REG)])
            ngrp = (hi - lo) // GG
            npair = ngrp // 2

            def start_idx(g, b):
                idx, sct, land, si, sg, ss = bufs[b]
                c1 = pltpu.async_copy(src_hbm.at[pl.ds(g, GG)], idx, si)
                c2 = pltpu.async_copy(sct_hbm.at[pl.ds(g, GG)], sct, si)
                return c1, c2

            def start_gather(b):
                idx, sct, land, si, sg, ss = bufs[b]
                return pltpu.async_copy(h_hbm.at[idx], land, sg)

            def start_scat(b):
                idx, sct, land, si, sg, ss = bufs[b]
                return pltpu.async_copy(land, acc_s.at[sct], ss, add=True)

            def pbody(k2, _, lo=lo):
                g0 = pl.multiple_of(lo + (2 * k2) * GG, GG)
                g1 = pl.multiple_of(g0 + GG, GG)
                i0a, i0b = start_idx(g0, 0)
                i1a, i1b = start_idx(g1, 1)
                i0a.wait()
                i0b.wait()
                gc0 = start_gather(0)
                i1a.wait()
                i1b.wait()
                gc1 = start_gather(1)
                gc0.wait()
                sc0 = start_scat(0)
                gc1.wait()
                sc0.wait()
                sc1 = start_scat(1)
                sc1.wait()
                return 0

            lax.fori_loop(0, npair, pbody, 0)

            @pl.when(ngrp % 2 == 1)
            def _():
                g = pl.multiple_of(lo + (ngrp - 1) * GG, GG)
                ia, ib = start_idx(g, 0)
                ia.wait()
                ib.wait()
                gc = start_gather(0)
                gc.wait()
                sc = start_scat(0)
                sc.wait()

            pltpu.sync_copy(acc_s.at[pl.ds(rbase, CH)],
                            out_hbm.at[pl.ds(chunk * CH, CH)])
            return 0

        lax.fori_loop(0, CPT, cbody, 0)

    return k(h2.reshape(N, SL, 128), src_p, sct_p, eo2,
             zbuf.reshape(REG, SL, 128))


def _cs(v, r):
    return v[:, r * FP:(r + 1) * FP]


def _k0_body(x_ref, d_ref, h_ref, xm_ref):
    x = x_ref[...]
    d = d_ref[...]
    acc = jnp.zeros((BN_, FP), jnp.float32)
    for r in range(R):
        h = x * (1.0 - d[:, r:r + 1])
        h_ref[:, r * FP:(r + 1) * FP] = h
        acc += h
    xm_ref[...] = acc * (1.0 / R)


def _init_h(xpad, dropT):
    return pl.pallas_call(
        _k0_body,
        grid=(NB,),
        in_specs=[
            pl.BlockSpec((BN_, FP), lambda nb: (nb, 0)),
            pl.BlockSpec((BN_, R), lambda nb: (nb, 0)),
        ],
        out_specs=[
            pl.BlockSpec((BN_, RW), lambda nb: (nb, 0)),
            pl.BlockSpec((BN_, FP), lambda nb: (nb, 0)),
        ],
        out_shape=[
            jax.ShapeDtypeStruct((N, RW), jnp.float32),
            jax.ShapeDtypeStruct((N, FP), jnp.float32),
        ],
    )(xpad, dropT)


def _k1_body(m_ref, w_ref, b_ref, z_ref, s1_ref, s2_ref):
    first = pl.program_id(0) == 0
    m = m_ref[...]
    w = w_ref[...]
    b = b_ref[...]
    s1 = jnp.zeros((1, FP), jnp.float32)
    s2 = jnp.zeros((1, FP), jnp.float32)
    for r in range(R):
        z = jnp.dot(_cs(m, r), w, preferred_element_type=jnp.float32) + b
        z_ref[:, r * FP:(r + 1) * FP] = z
        s1 += jnp.sum(z, axis=0, keepdims=True)
        s2 += jnp.sum(z * z, axis=0, keepdims=True)

    @pl.when(first)
    def _():
        s1_ref[...] = jnp.zeros_like(s1_ref)
        s2_ref[...] = jnp.zeros_like(s2_ref)
    s1_ref[...] += s1
    s2_ref[...] += s2


def _mm1_stats(msg, w, b):
    vec = pl.BlockSpec((1, FP), lambda nb: (0, 0))
    return pl.pallas_call(
        _k1_body,
        grid=(NB,),
        in_specs=[
            pl.BlockSpec((BN_, RW), lambda nb: (nb, 0)),
            pl.BlockSpec((FP, FP), lambda nb: (0, 0)),
            vec,
        ],
        out_specs=[pl.BlockSpec((BN_, RW), lambda nb: (nb, 0)), vec, vec],
        out_shape=[
            jax.ShapeDtypeStruct((N, RW), jnp.float32),
            jax.ShapeDtypeStruct((1, FP), jnp.float32),
            jax.ShapeDtypeStruct((1, FP), jnp.float32),
        ],
    )(msg, w, b)


def _bn_affine(s1, s2, g, bt):
    mu = s1 * (1.0 / M)
    var = s2 * (1.0 / M) - mu * mu
    a = g * lax.rsqrt(var + EPS)
    c = bt - mu * a
    return a, c


def _k2_body(z_ref, s1_ref, s2_ref, g_ref, bt_ref, w_ref, b_ref,
             z2_ref, t1_ref, t2_ref):
    first = pl.program_id(0) == 0
    a, c = _bn_affine(s1_ref[...], s2_ref[...], g_ref[...], bt_ref[...])
    z = z_ref[...]
    w = w_ref[...]
    b = b_ref[...]
    t1 = jnp.zeros((1, FP), jnp.float32)
    t2 = jnp.zeros((1, FP), jnp.float32)
    for r in range(R):
        zn = jnp.maximum(_cs(z, r) * a + c, 0.0)
        z2 = jnp.dot(zn, w, preferred_element_type=jnp.float32) + b
        z2_ref[:, r * FP:(r + 1) * FP] = z2
        t1 += jnp.sum(z2, axis=0, keepdims=True)
        t2 += jnp.sum(z2 * z2, axis=0, keepdims=True)

    @pl.when(first)
    def _():
        t1_ref[...] = jnp.zeros_like(t1_ref)
        t2_ref[...] = jnp.zeros_like(t2_ref)
    t1_ref[...] += t1
    t2_ref[...] += t2


def _bn_relu_mm2(z1, s1, s2, g, bt, w, b):
    vec = pl.BlockSpec((1, FP), lambda nb: (0, 0))
    return pl.pallas_call(
        _k2_body,
        grid=(NB,),
        in_specs=[
            pl.BlockSpec((BN_, RW), lambda nb: (nb, 0)),
            vec, vec, vec, vec,
            pl.BlockSpec((FP, FP), lambda nb: (0, 0)),
            vec,
        ],
        out_specs=[pl.BlockSpec((BN_, RW), lambda nb: (nb, 0)), vec, vec],
        out_shape=[
            jax.ShapeDtypeStruct((N, RW), jnp.float32),
            jax.ShapeDtypeStruct((1, FP), jnp.float32),
            jax.ShapeDtypeStruct((1, FP), jnp.float32),
        ],
    )(z1, s1, s2, g, bt, w, b)


def _k3_body(z_ref, s1_ref, s2_ref, g_ref, bt_ref, h_ref, xm_ref):
    a, c = _bn_affine(s1_ref[...], s2_ref[...], g_ref[...], bt_ref[...])
    z = z_ref[...]
    acc = jnp.zeros((BN_, FP), jnp.float32)
    for r in range(R):
        h = jnp.maximum(_cs(z, r) * a + c, 0.0)
        h_ref[:, r * FP:(r + 1) * FP] = h
        acc += h
    xm_ref[...] = acc * (1.0 / R)


def _bn_relu_out(z2, s1, s2, g, bt):
    vec = pl.BlockSpec((1, FP), lambda nb: (0, 0))
    return pl.pallas_call(
        _k3_body,
        grid=(NB,),
        in_specs=[
            pl.BlockSpec((BN_, RW), lambda nb: (nb, 0)),
            vec, vec, vec, vec,
        ],
        out_specs=[
            pl.BlockSpec((BN_, RW), lambda nb: (nb, 0)),
            pl.BlockSpec((BN_, FP), lambda nb: (nb, 0)),
        ],
        out_shape=[
            jax.ShapeDtypeStruct((N, RW), jnp.float32),
            jax.ShapeDtypeStruct((N, FP), jnp.float32),
        ],
    )(z2, s1, s2, g, bt)


def _k4a_body(b_ref, x0, x1, x2, x3, x4, p_ref, cnt_ref):
    first = pl.program_id(0) == 0
    gids = lax.broadcasted_iota(jnp.int32, (1, NG), 1)
    onehot = (b_ref[...] == gids).astype(jnp.float32)

    @pl.when(first)
    def _():
        p_ref[...] = jnp.zeros_like(p_ref)
        cnt_ref[...] = jnp.zeros_like(cnt_ref)
    cnt_ref[...] += jnp.sum(onehot, axis=0)[:, None]
    dn = (((0,), (0,)), ((), ()))
    for l, xr in enumerate((x0, x1, x2, x3, x4)):
        p_ref[l] += lax.dot_general(onehot, xr[...], dn,
                                    preferred_element_type=jnp.float32)


def _pool_accum(batch2, xms):
    blk = pl.BlockSpec((BN_, FP), lambda nb: (nb, 0))
    return pl.pallas_call(
        _k4a_body,
        grid=(NB,),
        in_specs=[pl.BlockSpec((BN_, 1), lambda nb: (nb, 0))] + [blk] * 5,
        out_specs=[
            pl.BlockSpec((5, NG, FP), lambda nb: (0, 0, 0)),
            pl.BlockSpec((NG, 1), lambda nb: (0, 0)),
        ],
        out_shape=[
            jax.ShapeDtypeStruct((5, NG, FP), jnp.float32),
            jax.ShapeDtypeStruct((NG, 1), jnp.float32),
        ],
    )(batch2, *xms)


def _k4b_body(p_ref, cnt_ref, w_ref, b_ref, o_ref):
    inv = 1.0 / jnp.maximum(cnt_ref[...], 1.0)
    acc = jnp.broadcast_to(b_ref[...], (NG, 128))
    for l in range(5):
        acc += jnp.dot(p_ref[l] * inv, w_ref[l],
                       preferred_element_type=jnp.float32)
    o_ref[...] = acc


def _pool_final(p, cnt, w, b):
    return pl.pallas_call(
        _k4b_body,
        out_shape=jax.ShapeDtypeStruct((NG, 128), jnp.float32),
    )(p, cnt, w, b)


def kernel(x, edge_index, batch, drop, params):
    f32 = jnp.float32

    def padw(w, co=FP):
        out = jnp.zeros((FP, co), f32)
        return out.at[: w.shape[0], : w.shape[1]].set(w)

    def padv(v):
        out = jnp.zeros((1, FP), f32)
        return out.at[0, : v.shape[0]].set(v)

    iota_n = jnp.arange(N, dtype=jnp.int32)
    src = jnp.concatenate([edge_index[0], iota_n])
    dst = jnp.concatenate([edge_index[1], iota_n])
    dst_s, src_s = lax.sort_key_val(dst, src)
    cid = dst_s // CH
    bounds = jnp.arange(NCH + 1, dtype=jnp.int32) * CH
    eo = jnp.searchsorted(dst_s, bounds, side="left").astype(jnp.int32)
    cnt = eo[1:] - eo[:-1]
    plen = ((cnt + GSZ - 1) // GSZ) * GSZ
    ps = jnp.concatenate([jnp.zeros((1,), jnp.int32),
                          jnp.cumsum(plen, dtype=jnp.int32)])
    pos = ps[cid] + (jnp.arange(EE, dtype=jnp.int32) - eo[cid])
    w_e = cid // CPT
    row_e = (w_e // 2) * REG + dst_s % CH
    src_p = jnp.zeros((PE,), jnp.int32).at[pos].set(src_s)
    sct_p = jnp.full((PE,), CH, jnp.int32).at[pos].set(row_e)
    eo2 = (jnp.zeros((NCH, 16), jnp.int32)
           .at[:, 0].set(ps[:-1]).at[:, 1].set(ps[1:]))
    zbuf = jnp.zeros((REG, RW), f32)

    xpad = jnp.zeros((N, FP), f32).at[:, :F].set(x)
    dropT = drop.astype(f32).T
    batch2 = batch[:, None]

    h, xm0 = _init_h(xpad, dropT)
    xms = [xm0]
    for i in range(NL):
        p = params["convs"][i]
        w1, b1 = padw(p["W1"]), padv(p["b1"])
        g1, bt1 = padv(p["g1"]), padv(p["bt1"])
        w2, b2 = padw(p["W2"]), padv(p["b2"])
        g2, bt2 = padv(params["bns"][i]["g"]), padv(params["bns"][i]["b"])

        msg_p = _sc_segsum(h, src_p, sct_p, eo2, zbuf).reshape(NP, RW)
        z1, s1, s2 = _mm1_stats(msg_p, w1, b1)
        z2, t1, t2 = _bn_relu_mm2(z1, s1, s2, g1, bt1, w2, b2)
        h, xm = _bn_relu_out(z2, t1, t2, g2, bt2)
        xms.append(xm)

    fw = jnp.stack([padw(params["fcs"][l]["W"], 128) for l in range(5)])
    fb = padv(sum(params["fcs"][l]["b"] for l in range(5)))[:, :128]
    p_acc, cnt = _pool_accum(batch2, xms)
    out = _pool_final(p_acc, cnt, fw, fb)
    return out[:, :10]


# speedup vs baseline: 1.0610x; 1.0610x over previous
import functools

import jax
import jax.numpy as jnp
from jax import lax
from jax.experimental import pallas as pl
from jax.experimental.pallas import tpu as pltpu, tpu_sc as plsc

R = 8
N = 10000
F = 300
FP = 384
RW = R * FP
E = 160000
NG = 64
NL = 4
EPS = 1e-5
M = R * N

CH = 16
CPT = 20
NTILES = 32
NCH = NTILES * CPT
NP = NCH * CH
GSZ = 16

NB = 25
BN_ = N // NB


REG = 24
PE = E + NCH * GSZ
SL = RW // 128
GG = 8


def _sc_segsum(h2, src_p, sct_p, eo2, zbuf):
    mesh = plsc.VectorSubcoreMesh(core_axis_name="c", subcore_axis_name="s")

    @functools.partial(
        pl.kernel,
        out_type=jax.ShapeDtypeStruct((NP, SL, 128), jnp.float32),
        mesh=mesh,
        scratch_types=[
            pltpu.VMEM((16,), jnp.int32),
            pltpu.VMEM((GG,), jnp.int32),
            pltpu.VMEM((GG,), jnp.int32),
            pltpu.VMEM((GG,), jnp.int32),
            pltpu.VMEM((GG,), jnp.int32),
            pltpu.VMEM((GG, SL, 128), jnp.float32),
            pltpu.VMEM((GG, SL, 128), jnp.float32),
            pltpu.VMEM_SHARED((16 * REG, SL, 128), jnp.float32),
            pltpu.SemaphoreType.DMA,
            pltpu.SemaphoreType.DMA,
            pltpu.SemaphoreType.DMA,
        ],
    )
    def k(h_hbm, src_hbm, sct_hbm, eo_hbm, z_hbm, out_hbm,
          eo_v, idx0, idx1, sct0, sct1, land0, land1, acc_s,
          si0, sg0, ss0):
        si1, sg1, ss1 = si0, sg0, ss0
        sid = lax.axis_index("s")
        wid = sid * 2 + lax.axis_index("c")
        rbase = sid * REG
        bufs = ((idx0, sct0, land0, si0, sg0, ss0),
                (idx1, sct1, land1, si1, sg1, ss1))

        def cbody(c, _):
            chunk = wid * CPT + c
            pltpu.sync_copy(eo_hbm.at[chunk], eo_v)
            eo_vec = eo_v[...]
            lo = pl.multiple_of(eo_vec[0], GSZ)
            hi = eo_vec[1]
            pltpu.sync_copy(z_hbm, acc_s.at[pl.ds(rbase, REG)])
            ngrp = (hi - lo) // GG
            npair = ngrp // 2

            def start_idx(g, b):
                idx, sct, land, si, sg, ss = bufs[b]
                c1 = pltpu.async_copy(src_hbm.at[pl.ds(g, GG)], idx, si)
                c2 = pltpu.async_copy(sct_hbm.at[pl.ds(g, GG)], sct, si)
                return c1, c2

            def start_gather(b):
                idx, sct, land, si, sg, ss = bufs[b]
                return pltpu.async_copy(h_hbm.at[idx], land, sg)

            def start_scat(b):
                idx, sct, land, si, sg, ss = bufs[b]
                return pltpu.async_copy(land, acc_s.at[sct], ss, add=True)

            def pbody(k2, _, lo=lo):
                g0 = pl.multiple_of(lo + (2 * k2) * GG, GG)
                g1 = pl.multiple_of(g0 + GG, GG)
                i0a, i0b = start_idx(g0, 0)
                i1a, i1b = start_idx(g1, 1)
                i0a.wait()
                i0b.wait()
                gc0 = start_gather(0)
                i1a.wait()
                i1b.wait()
                gc1 = start_gather(1)
                gc0.wait()
                sc0 = start_scat(0)
                gc1.wait()
                sc0.wait()
                sc1 = start_scat(1)
                sc1.wait()
                return 0

            lax.fori_loop(0, npair, pbody, 0)

            @pl.when(ngrp % 2 == 1)
            def _():
                g = pl.multiple_of(lo + (ngrp - 1) * GG, GG)
                ia, ib = start_idx(g, 0)
                ia.wait()
                ib.wait()
                gc = start_gather(0)
                gc.wait()
                sc = start_scat(0)
                sc.wait()

            pltpu.sync_copy(acc_s.at[pl.ds(rbase, CH)],
                            out_hbm.at[pl.ds(chunk * CH, CH)])
            return 0

        lax.fori_loop(0, CPT, cbody, 0)

    return k(h2.reshape(N, SL, 128), src_p, sct_p, eo2,
             zbuf.reshape(REG, SL, 128))


def _cs(v, r):
    return v[:, r * FP:(r + 1) * FP]


def _k0_body(x_ref, d_ref, h_ref, xm_ref):
    x = x_ref[...]
    d = d_ref[...]
    acc = jnp.zeros((BN_, FP), jnp.float32)
    for r in range(R):
        h = x * (1.0 - d[:, r:r + 1])
        h_ref[:, r * FP:(r + 1) * FP] = h
        acc += h
    xm_ref[...] = acc * (1.0 / R)


def _init_h(xpad, dropT):
    return pl.pallas_call(
        _k0_body,
        grid=(NB,),
        in_specs=[
            pl.BlockSpec((BN_, FP), lambda nb: (nb, 0)),
            pl.BlockSpec((BN_, R), lambda nb: (nb, 0)),
        ],
        out_specs=[
            pl.BlockSpec((BN_, RW), lambda nb: (nb, 0)),
            pl.BlockSpec((BN_, FP), lambda nb: (nb, 0)),
        ],
        out_shape=[
            jax.ShapeDtypeStruct((N, RW), jnp.float32),
            jax.ShapeDtypeStruct((N, FP), jnp.float32),
        ],
    )(xpad, dropT)


def _k1_body(h_ref, m_ref, w_ref, b_ref, z_ref, s1_ref, s2_ref):
    first = pl.program_id(0) == 0
    h = h_ref[...]
    m = m_ref[...]
    w = w_ref[...]
    b = b_ref[...]
    s1 = jnp.zeros((1, FP), jnp.float32)
    s2 = jnp.zeros((1, FP), jnp.float32)
    for r in range(R):
        z = jnp.dot(_cs(h, r) + _cs(m, r), w,
                    preferred_element_type=jnp.float32) + b
        z_ref[:, r * FP:(r + 1) * FP] = z
        s1 += jnp.sum(z, axis=0, keepdims=True)
        s2 += jnp.sum(z * z, axis=0, keepdims=True)

    @pl.when(first)
    def _():
        s1_ref[...] = jnp.zeros_like(s1_ref)
        s2_ref[...] = jnp.zeros_like(s2_ref)
    s1_ref[...] += s1
    s2_ref[...] += s2


def _mm1_stats(h, msg, w, b):
    vec = pl.BlockSpec((1, FP), lambda nb: (0, 0))
    return pl.pallas_call(
        _k1_body,
        grid=(NB,),
        in_specs=[
            pl.BlockSpec((BN_, RW), lambda nb: (nb, 0)),
            pl.BlockSpec((BN_, RW), lambda nb: (nb, 0)),
            pl.BlockSpec((FP, FP), lambda nb: (0, 0)),
            vec,
        ],
        out_specs=[pl.BlockSpec((BN_, RW), lambda nb: (nb, 0)), vec, vec],
        out_shape=[
            jax.ShapeDtypeStruct((N, RW), jnp.float32),
            jax.ShapeDtypeStruct((1, FP), jnp.float32),
            jax.ShapeDtypeStruct((1, FP), jnp.float32),
        ],
    )(h, msg, w, b)


def _bn_affine(s1, s2, g, bt):
    mu = s1 * (1.0 / M)
    var = s2 * (1.0 / M) - mu * mu
    a = g * lax.rsqrt(var + EPS)
    c = bt - mu * a
    return a, c


def _k2_body(z_ref, s1_ref, s2_ref, g_ref, bt_ref, w_ref, b_ref,
             z2_ref, t1_ref, t2_ref):
    first = pl.program_id(0) == 0
    a, c = _bn_affine(s1_ref[...], s2_ref[...], g_ref[...], bt_ref[...])
    z = z_ref[...]
    w = w_ref[...]
    b = b_ref[...]
    t1 = jnp.zeros((1, FP), jnp.float32)
    t2 = jnp.zeros((1, FP), jnp.float32)
    for r in range(R):
        zn = jnp.maximum(_cs(z, r) * a + c, 0.0)
        z2 = jnp.dot(zn, w, preferred_element_type=jnp.float32) + b
        z2_ref[:, r * FP:(r + 1) * FP] = z2
        t1 += jnp.sum(z2, axis=0, keepdims=True)
        t2 += jnp.sum(z2 * z2, axis=0, keepdims=True)

    @pl.when(first)
    def _():
        t1_ref[...] = jnp.zeros_like(t1_ref)
        t2_ref[...] = jnp.zeros_like(t2_ref)
    t1_ref[...] += t1
    t2_ref[...] += t2


def _bn_relu_mm2(z1, s1, s2, g, bt, w, b):
    vec = pl.BlockSpec((1, FP), lambda nb: (0, 0))
    return pl.pallas_call(
        _k2_body,
        grid=(NB,),
        in_specs=[
            pl.BlockSpec((BN_, RW), lambda nb: (nb, 0)),
            vec, vec, vec, vec,
            pl.BlockSpec((FP, FP), lambda nb: (0, 0)),
            vec,
        ],
        out_specs=[pl.BlockSpec((BN_, RW), lambda nb: (nb, 0)), vec, vec],
        out_shape=[
            jax.ShapeDtypeStruct((N, RW), jnp.float32),
            jax.ShapeDtypeStruct((1, FP), jnp.float32),
            jax.ShapeDtypeStruct((1, FP), jnp.float32),
        ],
    )(z1, s1, s2, g, bt, w, b)


def _k3_body(z_ref, s1_ref, s2_ref, g_ref, bt_ref, h_ref, xm_ref):
    a, c = _bn_affine(s1_ref[...], s2_ref[...], g_ref[...], bt_ref[...])
    z = z_ref[...]
    acc = jnp.zeros((BN_, FP), jnp.float32)
    for r in range(R):
        h = jnp.maximum(_cs(z, r) * a + c, 0.0)
        h_ref[:, r * FP:(r + 1) * FP] = h
        acc += h
    xm_ref[...] = acc * (1.0 / R)


def _bn_relu_out(z2, s1, s2, g, bt):
    vec = pl.BlockSpec((1, FP), lambda nb: (0, 0))
    return pl.pallas_call(
        _k3_body,
        grid=(NB,),
        in_specs=[
            pl.BlockSpec((BN_, RW), lambda nb: (nb, 0)),
            vec, vec, vec, vec,
        ],
        out_specs=[
            pl.BlockSpec((BN_, RW), lambda nb: (nb, 0)),
            pl.BlockSpec((BN_, FP), lambda nb: (nb, 0)),
        ],
        out_shape=[
            jax.ShapeDtypeStruct((N, RW), jnp.float32),
            jax.ShapeDtypeStruct((N, FP), jnp.float32),
        ],
    )(z2, s1, s2, g, bt)


def _k4a_body(b_ref, x0, x1, x2, x3, x4, p_ref, cnt_ref):
    first = pl.program_id(0) == 0
    gids = lax.broadcasted_iota(jnp.int32, (1, NG), 1)
    onehot = (b_ref[...] == gids).astype(jnp.float32)

    @pl.when(first)
    def _():
        p_ref[...] = jnp.zeros_like(p_ref)
        cnt_ref[...] = jnp.zeros_like(cnt_ref)
    cnt_ref[...] += jnp.sum(onehot, axis=0)[:, None]
    dn = (((0,), (0,)), ((), ()))
    for l, xr in enumerate((x0, x1, x2, x3, x4)):
        p_ref[l] += lax.dot_general(onehot, xr[...], dn,
                                    preferred_element_type=jnp.float32)


def _pool_accum(batch2, xms):
    blk = pl.BlockSpec((BN_, FP), lambda nb: (nb, 0))
    return pl.pallas_call(
        _k4a_body,
        grid=(NB,),
        in_specs=[pl.BlockSpec((BN_, 1), lambda nb: (nb, 0))] + [blk] * 5,
        out_specs=[
            pl.BlockSpec((5, NG, FP), lambda nb: (0, 0, 0)),
            pl.BlockSpec((NG, 1), lambda nb: (0, 0)),
        ],
        out_shape=[
            jax.ShapeDtypeStruct((5, NG, FP), jnp.float32),
            jax.ShapeDtypeStruct((NG, 1), jnp.float32),
        ],
    )(batch2, *xms)


def _k4b_body(p_ref, cnt_ref, w_ref, b_ref, o_ref):
    inv = 1.0 / jnp.maximum(cnt_ref[...], 1.0)
    acc = jnp.broadcast_to(b_ref[...], (NG, 128))
    for l in range(5):
        acc += jnp.dot(p_ref[l] * inv, w_ref[l],
                       preferred_element_type=jnp.float32)
    o_ref[...] = acc


def _pool_final(p, cnt, w, b):
    return pl.pallas_call(
        _k4b_body,
        out_shape=jax.ShapeDtypeStruct((NG, 128), jnp.float32),
    )(p, cnt, w, b)


def kernel(x, edge_index, batch, drop, params):
    f32 = jnp.float32

    def padw(w, co=FP):
        out = jnp.zeros((FP, co), f32)
        return out.at[: w.shape[0], : w.shape[1]].set(w)

    def padv(v):
        out = jnp.zeros((1, FP), f32)
        return out.at[0, : v.shape[0]].set(v)

    src, dst = edge_index[0], edge_index[1]
    dst_s, src_s = lax.sort_key_val(dst, src)
    cid = dst_s // CH
    bounds = jnp.arange(NCH + 1, dtype=jnp.int32) * CH
    eo = jnp.searchsorted(dst_s, bounds, side="left").astype(jnp.int32)
    cnt = eo[1:] - eo[:-1]
    plen = ((cnt + GSZ - 1) // GSZ) * GSZ
    ps = jnp.concatenate([jnp.zeros((1,), jnp.int32),
                          jnp.cumsum(plen, dtype=jnp.int32)])
    pos = ps[cid] + (jnp.arange(E, dtype=jnp.int32) - eo[cid])
    w_e = cid // CPT
    row_e = (w_e // 2) * REG + dst_s % CH
    src_p = jnp.zeros((PE,), jnp.int32).at[pos].set(src_s)
    sct_p = jnp.full((PE,), CH, jnp.int32).at[pos].set(row_e)
    eo2 = (jnp.zeros((NCH, 16), jnp.int32)
           .at[:, 0].set(ps[:-1]).at[:, 1].set(ps[1:]))
    zbuf = jnp.zeros((REG, RW), f32)

    xpad = jnp.zeros((N, FP), f32).at[:, :F].set(x)
    dropT = drop.astype(f32).T
    batch2 = batch[:, None]

    h, xm0 = _init_h(xpad, dropT)
    xms = [xm0]
    for i in range(NL):
        p = params["convs"][i]
        w1, b1 = padw(p["W1"]), padv(p["b1"])
        g1, bt1 = padv(p["g1"]), padv(p["bt1"])
        w2, b2 = padw(p["W2"]), padv(p["b2"])
        g2, bt2 = padv(params["bns"][i]["g"]), padv(params["bns"][i]["b"])

        msg_p = _sc_segsum(h, src_p, sct_p, eo2, zbuf).reshape(NP, RW)
        z1, s1, s2 = _mm1_stats(h, msg_p, w1, b1)
        z2, t1, t2 = _bn_relu_mm2(z1, s1, s2, g1, bt1, w2, b2)
        h, xm = _bn_relu_out(z2, t1, t2, g2, bt2)
        xms.append(xm)

    fw = jnp.stack([padw(params["fcs"][l]["W"], 128) for l in range(5)])
    fb = padv(sum(params["fcs"][l]["b"] for l in range(5)))[:, :128]
    p_acc, cnt = _pool_accum(batch2, xms)
    out = _pool_final(p_acc, cnt, fw, fb)
    return out[:, :10]


# speedup vs baseline: 1.1524x; 1.0861x over previous
import functools

import jax
import jax.numpy as jnp
from jax import lax
from jax.experimental import pallas as pl
from jax.experimental.pallas import tpu as pltpu, tpu_sc as plsc

R = 8
N = 10000
F = 300
FP = 384
RW = R * FP
E = 160000
NG = 64
NL = 4
EPS = 1e-5
M = R * N

CH = 16
CPT = 20
NTILES = 32
NCH = NTILES * CPT
NP = NCH * CH
GSZ = 16

NB = 25
BN_ = N // NB


REG = 24
PE = E + NCH * GSZ
SL = RW // 128
GG = 8


def _sc_segsum(h2, src_p, sct_p, eo2, zbuf):
    mesh = plsc.VectorSubcoreMesh(core_axis_name="c", subcore_axis_name="s")

    @functools.partial(
        pl.kernel,
        out_type=jax.ShapeDtypeStruct((NP, SL, 128), jnp.float32),
        mesh=mesh,
        scratch_types=[
            pltpu.VMEM((16,), jnp.int32),
            pltpu.VMEM((GG,), jnp.int32),
            pltpu.VMEM((GG,), jnp.int32),
            pltpu.VMEM((GG,), jnp.int32),
            pltpu.VMEM((GG,), jnp.int32),
            pltpu.VMEM((GG, SL, 128), jnp.float32),
            pltpu.VMEM((GG, SL, 128), jnp.float32),
            pltpu.VMEM_SHARED((16 * REG, SL, 128), jnp.float32),
            pltpu.SemaphoreType.DMA,
            pltpu.SemaphoreType.DMA,
            pltpu.SemaphoreType.DMA,
            pltpu.SemaphoreType.DMA,
            pltpu.SemaphoreType.DMA,
            pltpu.SemaphoreType.DMA,
        ],
    )
    def k(h_hbm, src_hbm, sct_hbm, eo_hbm, z_hbm, out_hbm,
          eo_v, idx0, idx1, sct0, sct1, land0, land1, acc_s,
          si0, si1, sg0, sg1, ss0, ss1):
        sid = lax.axis_index("s")
        wid = sid * 2 + lax.axis_index("c")
        rbase = sid * REG
        bufs = ((idx0, sct0, land0, si0, sg0, ss0),
                (idx1, sct1, land1, si1, sg1, ss1))

        def cbody(c, _):
            chunk = wid * CPT + c
            pltpu.sync_copy(eo_hbm.at[chunk], eo_v)
            eo_vec = eo_v[...]
            lo = pl.multiple_of(eo_vec[0], GSZ)
            hi = eo_vec[1]
            pltpu.sync_copy(z_hbm, acc_s.at[pl.ds(rbase, REG)])
            ngrp = (hi - lo) // GG
            npair = ngrp // 2

            def start_idx(g, b):
                idx, sct, land, si, sg, ss = bufs[b]
                c1 = pltpu.async_copy(src_hbm.at[pl.ds(g, GG)], idx, si)
                c2 = pltpu.async_copy(sct_hbm.at[pl.ds(g, GG)], sct, si)
                return c1, c2

            def start_gather(b):
                idx, sct, land, si, sg, ss = bufs[b]
                return pltpu.async_copy(h_hbm.at[idx], land, sg)

            def start_scat(b):
                idx, sct, land, si, sg, ss = bufs[b]
                return pltpu.async_copy(land, acc_s.at[sct], ss, add=True)

            def drain_scat1():
                pltpu.make_async_copy(land1, acc_s.at[sct1], ss1).wait()

            def pbody(k2, _, lo=lo):
                g0 = pl.multiple_of(lo + (2 * k2) * GG, GG)
                g1 = pl.multiple_of(g0 + GG, GG)
                i0a, i0b = start_idx(g0, 0)
                i0a.wait()
                i0b.wait()
                gc0 = start_gather(0)

                @pl.when(k2 > 0)
                def _():
                    drain_scat1()

                i1a, i1b = start_idx(g1, 1)
                i1a.wait()
                i1b.wait()
                gc1 = start_gather(1)
                gc0.wait()
                sc0 = start_scat(0)
                gc1.wait()
                sc0.wait()
                start_scat(1)
                return 0

            lax.fori_loop(0, npair, pbody, 0)

            @pl.when(npair > 0)
            def _():
                drain_scat1()

            @pl.when(ngrp % 2 == 1)
            def _():
                g = pl.multiple_of(lo + (ngrp - 1) * GG, GG)
                ia, ib = start_idx(g, 0)
                ia.wait()
                ib.wait()
                gc = start_gather(0)
                gc.wait()
                sc = start_scat(0)
                sc.wait()

            pltpu.sync_copy(acc_s.at[pl.ds(rbase, CH)],
                            out_hbm.at[pl.ds(chunk * CH, CH)])
            return 0

        lax.fori_loop(0, CPT, cbody, 0)

    return k(h2.reshape(N, SL, 128), src_p, sct_p, eo2,
             zbuf.reshape(REG, SL, 128))


def _cs(v, r):
    return v[:, r * FP:(r + 1) * FP]


def _k0_body(x_ref, d_ref, h_ref, xm_ref):
    x = x_ref[...]
    d = d_ref[...]
    acc = jnp.zeros((BN_, FP), jnp.float32)
    for r in range(R):
        h = x * (1.0 - d[:, r:r + 1])
        h_ref[:, r * FP:(r + 1) * FP] = h
        acc += h
    xm_ref[...] = acc * (1.0 / R)


def _init_h(xpad, dropT):
    return pl.pallas_call(
        _k0_body,
        grid=(NB,),
        in_specs=[
            pl.BlockSpec((BN_, FP), lambda nb: (nb, 0)),
            pl.BlockSpec((BN_, R), lambda nb: (nb, 0)),
        ],
        out_specs=[
            pl.BlockSpec((BN_, RW), lambda nb: (nb, 0)),
            pl.BlockSpec((BN_, FP), lambda nb: (nb, 0)),
        ],
        out_shape=[
            jax.ShapeDtypeStruct((N, RW), jnp.float32),
            jax.ShapeDtypeStruct((N, FP), jnp.float32),
        ],
    )(xpad, dropT)


def _k1_body(h_ref, m_ref, w_ref, b_ref, z_ref, s1_ref, s2_ref):
    first = pl.program_id(0) == 0
    h = h_ref[...]
    m = m_ref[...]
    w = w_ref[...]
    b = b_ref[...]
    s1 = jnp.zeros((1, FP), jnp.float32)
    s2 = jnp.zeros((1, FP), jnp.float32)
    for r in range(R):
        z = jnp.dot(_cs(h, r) + _cs(m, r), w,
                    preferred_element_type=jnp.float32) + b
        z_ref[:, r * FP:(r + 1) * FP] = z
        s1 += jnp.sum(z, axis=0, keepdims=True)
        s2 += jnp.sum(z * z, axis=0, keepdims=True)

    @pl.when(first)
    def _():
        s1_ref[...] = jnp.zeros_like(s1_ref)
        s2_ref[...] = jnp.zeros_like(s2_ref)
    s1_ref[...] += s1
    s2_ref[...] += s2


def _mm1_stats(h, msg, w, b):
    vec = pl.BlockSpec((1, FP), lambda nb: (0, 0))
    return pl.pallas_call(
        _k1_body,
        grid=(NB,),
        in_specs=[
            pl.BlockSpec((BN_, RW), lambda nb: (nb, 0)),
            pl.BlockSpec((BN_, RW), lambda nb: (nb, 0)),
            pl.BlockSpec((FP, FP), lambda nb: (0, 0)),
            vec,
        ],
        out_specs=[pl.BlockSpec((BN_, RW), lambda nb: (nb, 0)), vec, vec],
        out_shape=[
            jax.ShapeDtypeStruct((N, RW), jnp.float32),
            jax.ShapeDtypeStruct((1, FP), jnp.float32),
            jax.ShapeDtypeStruct((1, FP), jnp.float32),
        ],
    )(h, msg, w, b)


def _bn_affine(s1, s2, g, bt):
    mu = s1 * (1.0 / M)
    var = s2 * (1.0 / M) - mu * mu
    a = g * lax.rsqrt(var + EPS)
    c = bt - mu * a
    return a, c


def _k2_body(z_ref, s1_ref, s2_ref, g_ref, bt_ref, w_ref, b_ref,
             z2_ref, t1_ref, t2_ref):
    first = pl.program_id(0) == 0
    a, c = _bn_affine(s1_ref[...], s2_ref[...], g_ref[...], bt_ref[...])
    z = z_ref[...]
    w = w_ref[...]
    b = b_ref[...]
    t1 = jnp.zeros((1, FP), jnp.float32)
    t2 = jnp.zeros((1, FP), jnp.float32)
    for r in range(R):
        zn = jnp.maximum(_cs(z, r) * a + c, 0.0)
        z2 = jnp.dot(zn, w, preferred_element_type=jnp.float32) + b
        z2_ref[:, r * FP:(r + 1) * FP] = z2
        t1 += jnp.sum(z2, axis=0, keepdims=True)
        t2 += jnp.sum(z2 * z2, axis=0, keepdims=True)

    @pl.when(first)
    def _():
        t1_ref[...] = jnp.zeros_like(t1_ref)
        t2_ref[...] = jnp.zeros_like(t2_ref)
    t1_ref[...] += t1
    t2_ref[...] += t2


def _bn_relu_mm2(z1, s1, s2, g, bt, w, b):
    vec = pl.BlockSpec((1, FP), lambda nb: (0, 0))
    return pl.pallas_call(
        _k2_body,
        grid=(NB,),
        in_specs=[
            pl.BlockSpec((BN_, RW), lambda nb: (nb, 0)),
            vec, vec, vec, vec,
            pl.BlockSpec((FP, FP), lambda nb: (0, 0)),
            vec,
        ],
        out_specs=[pl.BlockSpec((BN_, RW), lambda nb: (nb, 0)), vec, vec],
        out_shape=[
            jax.ShapeDtypeStruct((N, RW), jnp.float32),
            jax.ShapeDtypeStruct((1, FP), jnp.float32),
            jax.ShapeDtypeStruct((1, FP), jnp.float32),
        ],
    )(z1, s1, s2, g, bt, w, b)


def _k3_body(z_ref, s1_ref, s2_ref, g_ref, bt_ref, h_ref, xm_ref):
    a, c = _bn_affine(s1_ref[...], s2_ref[...], g_ref[...], bt_ref[...])
    z = z_ref[...]
    acc = jnp.zeros((BN_, FP), jnp.float32)
    for r in range(R):
        h = jnp.maximum(_cs(z, r) * a + c, 0.0)
        h_ref[:, r * FP:(r + 1) * FP] = h
        acc += h
    xm_ref[...] = acc * (1.0 / R)


def _bn_relu_out(z2, s1, s2, g, bt):
    vec = pl.BlockSpec((1, FP), lambda nb: (0, 0))
    return pl.pallas_call(
        _k3_body,
        grid=(NB,),
        in_specs=[
            pl.BlockSpec((BN_, RW), lambda nb: (nb, 0)),
            vec, vec, vec, vec,
        ],
        out_specs=[
            pl.BlockSpec((BN_, RW), lambda nb: (nb, 0)),
            pl.BlockSpec((BN_, FP), lambda nb: (nb, 0)),
        ],
        out_shape=[
            jax.ShapeDtypeStruct((N, RW), jnp.float32),
            jax.ShapeDtypeStruct((N, FP), jnp.float32),
        ],
    )(z2, s1, s2, g, bt)


def _k4a_body(b_ref, x0, x1, x2, x3, x4, p_ref, cnt_ref):
    first = pl.program_id(0) == 0
    gids = lax.broadcasted_iota(jnp.int32, (1, NG), 1)
    onehot = (b_ref[...] == gids).astype(jnp.float32)

    @pl.when(first)
    def _():
        p_ref[...] = jnp.zeros_like(p_ref)
        cnt_ref[...] = jnp.zeros_like(cnt_ref)
    cnt_ref[...] += jnp.sum(onehot, axis=0)[:, None]
    dn = (((0,), (0,)), ((), ()))
    for l, xr in enumerate((x0, x1, x2, x3, x4)):
        p_ref[l] += lax.dot_general(onehot, xr[...], dn,
                                    preferred_element_type=jnp.float32)


def _pool_accum(batch2, xms):
    blk = pl.BlockSpec((BN_, FP), lambda nb: (nb, 0))
    return pl.pallas_call(
        _k4a_body,
        grid=(NB,),
        in_specs=[pl.BlockSpec((BN_, 1), lambda nb: (nb, 0))] + [blk] * 5,
        out_specs=[
            pl.BlockSpec((5, NG, FP), lambda nb: (0, 0, 0)),
            pl.BlockSpec((NG, 1), lambda nb: (0, 0)),
        ],
        out_shape=[
            jax.ShapeDtypeStruct((5, NG, FP), jnp.float32),
            jax.ShapeDtypeStruct((NG, 1), jnp.float32),
        ],
    )(batch2, *xms)


def _k4b_body(p_ref, cnt_ref, w_ref, b_ref, o_ref):
    inv = 1.0 / jnp.maximum(cnt_ref[...], 1.0)
    acc = jnp.broadcast_to(b_ref[...], (NG, 128))
    for l in range(5):
        acc += jnp.dot(p_ref[l] * inv, w_ref[l],
                       preferred_element_type=jnp.float32)
    o_ref[...] = acc


def _pool_final(p, cnt, w, b):
    return pl.pallas_call(
        _k4b_body,
        out_shape=jax.ShapeDtypeStruct((NG, 128), jnp.float32),
    )(p, cnt, w, b)


def kernel(x, edge_index, batch, drop, params):
    f32 = jnp.float32

    def padw(w, co=FP):
        out = jnp.zeros((FP, co), f32)
        return out.at[: w.shape[0], : w.shape[1]].set(w)

    def padv(v):
        out = jnp.zeros((1, FP), f32)
        return out.at[0, : v.shape[0]].set(v)

    src, dst = edge_index[0], edge_index[1]
    dst_s, src_s = lax.sort_key_val(dst, src)
    cid = dst_s // CH
    bounds = jnp.arange(NCH + 1, dtype=jnp.int32) * CH
    eo = jnp.searchsorted(dst_s, bounds, side="left").astype(jnp.int32)
    cnt = eo[1:] - eo[:-1]
    plen = ((cnt + GSZ - 1) // GSZ) * GSZ
    ps = jnp.concatenate([jnp.zeros((1,), jnp.int32),
                          jnp.cumsum(plen, dtype=jnp.int32)])
    pos = ps[cid] + (jnp.arange(E, dtype=jnp.int32) - eo[cid])
    w_e = cid // CPT
    row_e = (w_e // 2) * REG + dst_s % CH
    src_p = jnp.zeros((PE,), jnp.int32).at[pos].set(src_s)
    sct_p = jnp.full((PE,), CH, jnp.int32).at[pos].set(row_e)
    eo2 = (jnp.zeros((NCH, 16), jnp.int32)
           .at[:, 0].set(ps[:-1]).at[:, 1].set(ps[1:]))
    zbuf = jnp.zeros((REG, RW), f32)

    xpad = jnp.zeros((N, FP), f32).at[:, :F].set(x)
    dropT = drop.astype(f32).T
    batch2 = batch[:, None]

    h, xm0 = _init_h(xpad, dropT)
    xms = [xm0]
    for i in range(NL):
        p = params["convs"][i]
        w1, b1 = padw(p["W1"]), padv(p["b1"])
        g1, bt1 = padv(p["g1"]), padv(p["bt1"])
        w2, b2 = padw(p["W2"]), padv(p["b2"])
        g2, bt2 = padv(params["bns"][i]["g"]), padv(params["bns"][i]["b"])

        msg_p = _sc_segsum(h, src_p, sct_p, eo2, zbuf).reshape(NP, RW)
        z1, s1, s2 = _mm1_stats(h, msg_p, w1, b1)
        z2, t1, t2 = _bn_relu_mm2(z1, s1, s2, g1, bt1, w2, b2)
        h, xm = _bn_relu_out(z2, t1, t2, g2, bt2)
        xms.append(xm)

    fw = jnp.stack([padw(params["fcs"][l]["W"], 128) for l in range(5)])
    fb = padv(sum(params["fcs"][l]["b"] for l in range(5)))[:, :128]
    p_acc, cnt = _pool_accum(batch2, xms)
    out = _pool_final(p_acc, cnt, fw, fb)
    return out[:, :10]
